# baseline (speedup 1.0000x reference)
import functools

import jax
import jax.numpy as jnp
from jax import lax
from jax.experimental import pallas as pl
from jax.experimental.pallas import tpu as pltpu
from jax.experimental.pallas import tpu_sc as plsc

N = 10000
D = 128
F = 64
K_HOPS = 10
NC = 2
NS = 16
NR = 10240
RPT = NR // NS
E = 320000
STEP = 128
B = 8
NBLK = 20
NSTEPS = NBLK * B
EP = NS * NSTEPS * STEP
CH = 32


def _mlp_body(x_ref, w1_ref, b1_ref, g1_ref, be1_ref,
              w2_ref, b2_ref, g2_ref, be2_ref, out_ref):
    eps = jnp.float32(1e-5)
    h = jnp.dot(x_ref[...], w1_ref[...],
                preferred_element_type=jnp.float32) + b1_ref[...]
    mean = jnp.mean(h, axis=0, keepdims=True)
    var = jnp.mean(jnp.square(h - mean), axis=0, keepdims=True)
    h = (h - mean) * lax.rsqrt(var + eps) * g1_ref[...] + be1_ref[...]
    h = jnp.maximum(h, 0.0)
    h = jnp.dot(h, w2_ref[...],
                preferred_element_type=jnp.float32) + b2_ref[...]
    mean = jnp.mean(h, axis=0, keepdims=True)
    var = jnp.mean(jnp.square(h - mean), axis=0, keepdims=True)
    h = (h - mean) * lax.rsqrt(var + eps) * g2_ref[...] + be2_ref[...]
    h = jnp.maximum(h, 0.0)
    zpad = jnp.zeros((NR - N, F), jnp.float32)
    out_ref[0, pl.ds(0, N), :] = h[:, :F]
    out_ref[1, pl.ds(0, N), :] = h[:, F:]
    out_ref[0, pl.ds(N, NR - N), :] = zpad
    out_ref[1, pl.ds(N, NR - N), :] = zpad


def _rsqrt16(x):
    i = lax.bitcast_convert_type(x, jnp.int32)
    i = jnp.int32(0x5F3759DF) - (i >> 1)
    y = lax.bitcast_convert_type(i, jnp.float32)
    for _ in range(3):
        y = y * (jnp.float32(1.5) - jnp.float32(0.5) * x * y * y)
    return y


def _sc_body(h0_hbm, srcp_hbm, dstp_hbm, out_hbm, u_hbm,
             acc_sh, ds_sh, w_sh, dd_sh,
             sidx_b, didx_b, rows4, ones_v, rbuf2, zbuf, mbuf2,
             tb_a, tb_b, tb_c, gsem, ssem, isem, jsem,
             rsem, msem, usem, zsem):
    c = lax.axis_index("c")
    s = lax.axis_index("s")
    r0 = s * RPT
    uc = u_hbm.at[c]

    z16 = jnp.zeros((16,), jnp.float32)

    def zrow(i, _):
        for k4 in range(F // 16):
            zbuf[i, pl.ds(k4 * 16, 16)] = z16
        return 0
    lax.fori_loop(0, CH, zrow, 0)

    def zvec(i, _):
        tb_a[pl.ds(i * 16, 16)] = z16
        return 0
    lax.fori_loop(0, RPT // 16, zvec, 0)

    def o16(i, _):
        ones_v[pl.ds(i * 16, 16)] = jnp.ones((16,), jnp.float32)
        return 0
    lax.fori_loop(0, STEP // 16, o16, 0)

    pltpu.sync_copy(tb_a, ds_sh.at[pl.ds(r0, RPT)])
    pltpu.sync_copy(tb_a, w_sh.at[pl.ds(r0, RPT)])
    plsc.subcore_barrier()

    pltpu.sync_copy(srcp_hbm.at[s, 0], sidx_b.at[0])
    pltpu.sync_copy(dstp_hbm.at[s, 0], didx_b.at[0])

    def dblock(b, _):
        bslot = lax.rem(b, 2)
        nbslot = 1 - bslot

        @pl.when(b >= 1)
        def _():
            pltpu.make_async_copy(srcp_hbm.at[s, b], sidx_b.at[bslot],
                                  isem).wait()
            pltpu.make_async_copy(dstp_hbm.at[s, b], didx_b.at[bslot],
                                  jsem).wait()
            for j in range(B):
                pltpu.make_async_copy(
                    ones_v, ds_sh.at[sidx_b.at[nbslot, j]],
                    gsem.at[nbslot]).wait()
                pltpu.make_async_copy(
                    ones_v, w_sh.at[didx_b.at[nbslot, j]],
                    ssem.at[nbslot]).wait()

        @pl.when(b + 1 < NBLK)
        def _():
            pltpu.async_copy(srcp_hbm.at[s, b + 1], sidx_b.at[nbslot],
                             isem)
            pltpu.async_copy(dstp_hbm.at[s, b + 1], didx_b.at[nbslot],
                             jsem)

        for j in range(B):
            pltpu.async_copy(ones_v, ds_sh.at[sidx_b.at[bslot, j]],
                             gsem.at[bslot], add=True)
            pltpu.async_copy(ones_v, w_sh.at[didx_b.at[bslot, j]],
                             ssem.at[bslot], add=True)
        return 0
    lax.fori_loop(0, NBLK, dblock, 0)
    lastd = (NBLK - 1) % 2
    for j in range(B):
        pltpu.make_async_copy(
            ones_v, ds_sh.at[sidx_b.at[lastd, j]], gsem.at[lastd]).wait()
        pltpu.make_async_copy(
            ones_v, w_sh.at[didx_b.at[lastd, j]], ssem.at[lastd]).wait()

    pltpu.sync_copy(h0_hbm.at[c, pl.ds(r0, RPT), :],
                    acc_sh.at[pl.ds(r0, RPT), :])
    plsc.subcore_barrier()

    pltpu.sync_copy(ds_sh.at[pl.ds(r0, RPT)], tb_a)
    pltpu.sync_copy(w_sh.at[pl.ds(r0, RPT)], tb_b)
    def nloop(i, _):
        sl = pl.ds(i * 16, 16)
        xo = jnp.maximum(tb_a[sl], 1.0)
        xi = jnp.maximum(tb_b[sl], 1.0)
        ys = _rsqrt16(xo)
        yd = _rsqrt16(xi)
        tb_a[sl] = ys
        tb_b[sl] = ys * yd
        tb_c[sl] = yd
        return 0
    lax.fori_loop(0, RPT // 16, nloop, 0)
    pltpu.sync_copy(tb_a, ds_sh.at[pl.ds(r0, RPT)])
    pltpu.sync_copy(tb_b, w_sh.at[pl.ds(r0, RPT)])
    pltpu.sync_copy(tb_c, dd_sh.at[pl.ds(r0, RPT)])

    NCHK = RPT // CH

    def _rescale(mul_ref, to_hbm):

        def _compute(slot, cb):
            pltpu.sync_copy(acc_sh.at[pl.ds(cb, CH), :], rbuf2.at[slot])
            pltpu.sync_copy(mul_ref.at[pl.ds(cb, CH)], mbuf2.at[slot])

            def rgroup(g, _):
                mv = mbuf2[slot, pl.ds(g * 16, 16)]
                for r16 in range(16):
                    row = g * 16 + r16
                    m = mv[r16]
                    for k4 in range(F // 16):
                        sl = pl.ds(k4 * 16, 16)
                        rbuf2[slot, row, sl] = rbuf2[slot, row, sl] * m
                return 0
            lax.fori_loop(0, CH // 16, rgroup, 0)

        if to_hbm:
            def chunk(i, _):
                cb = r0 + i * CH
                _compute(0, cb)

                @pl.when(cb + CH <= N)
                def _():
                    pltpu.sync_copy(
                        rbuf2.at[0],
                        out_hbm.at[pl.ds(cb, CH), pl.ds(c * F, F)])

                @pl.when(cb == N - 16)
                def _():
                    pltpu.sync_copy(
                        rbuf2.at[0, pl.ds(0, 16), :],
                        out_hbm.at[pl.ds(cb, 16), pl.ds(c * F, F)])
                return 0
            lax.fori_loop(0, NCHK, chunk, 0)
        else:
            def chunk(ii, _):
                for half in range(2):
                    i = 2 * ii + half
                    cb = r0 + i * CH

                    @pl.when(i >= 2)
                    def _():
                        pltpu.make_async_copy(
                            rbuf2.at[half],
                            uc.at[pl.ds(cb - 2 * CH, CH), :],
                            usem.at[half]).wait()

                    _compute(half, cb)
                    pltpu.async_copy(rbuf2.at[half],
                                     uc.at[pl.ds(cb, CH), :],
                                     usem.at[half])
                    pltpu.async_copy(zbuf, acc_sh.at[pl.ds(cb, CH), :],
                                     zsem)
                return 0
            lax.fori_loop(0, NCHK // 2, chunk, 0)
            for half in range(2):
                pltpu.make_async_copy(
                    rbuf2.at[half],
                    uc.at[pl.ds(r0 + (NCHK - 2 + half) * CH, CH), :],
                    usem.at[half]).wait()

            def zdrain(i, _):
                pltpu.make_async_copy(
                    zbuf, acc_sh.at[pl.ds(r0, CH), :], zsem).wait()
                return 0
            lax.fori_loop(0, NCHK, zdrain, 0)

    _rescale(ds_sh, False)
    plsc.subcore_barrier()

    def _edge_sweep():
        pltpu.sync_copy(srcp_hbm.at[s, 0], sidx_b.at[0])
        pltpu.sync_copy(dstp_hbm.at[s, 0], didx_b.at[0])
        for q in range(3):
            pltpu.async_copy(uc.at[sidx_b.at[0, q]], rows4.at[q],
                             gsem.at[q])

        def bloop(b, _):
            bslot = lax.rem(b, 2)
            nbslot = 1 - bslot
            for j in range(B):
                q = j % 4
                pltpu.make_async_copy(
                    uc.at[sidx_b.at[bslot, j]], rows4.at[q],
                    gsem.at[q]).wait()
                pltpu.async_copy(rows4.at[q],
                                 acc_sh.at[didx_b.at[bslot, j]],
                                 ssem.at[q], add=True)
                if j == 0:
                    @pl.when(b >= 1)
                    def _():
                        pltpu.make_async_copy(
                            rows4.at[3], acc_sh.at[didx_b.at[nbslot, 7]],
                            ssem.at[3]).wait()

                    @pl.when(b + 1 < NBLK)
                    def _():
                        pltpu.async_copy(srcp_hbm.at[s, b + 1],
                                         sidx_b.at[nbslot], isem)
                        pltpu.async_copy(dstp_hbm.at[s, b + 1],
                                         didx_b.at[nbslot], jsem)
                else:
                    pltpu.make_async_copy(
                        rows4.at[(j - 1) % 4],
                        acc_sh.at[didx_b.at[bslot, j - 1]],
                        ssem.at[(j - 1) % 4]).wait()
                if j < 5:
                    pltpu.async_copy(uc.at[sidx_b.at[bslot, j + 3]],
                                     rows4.at[(j + 3) % 4],
                                     gsem.at[(j + 3) % 4])
                elif j == 5:
                    @pl.when(b + 1 < NBLK)
                    def _():
                        pltpu.make_async_copy(srcp_hbm.at[s, b + 1],
                                              sidx_b.at[nbslot],
                                              isem).wait()
                        pltpu.make_async_copy(dstp_hbm.at[s, b + 1],
                                              didx_b.at[nbslot],
                                              jsem).wait()
                        pltpu.async_copy(uc.at[sidx_b.at[nbslot, 0]],
                                         rows4.at[0], gsem.at[0])
                elif j == 6:
                    @pl.when(b + 1 < NBLK)
                    def _():
                        pltpu.async_copy(uc.at[sidx_b.at[nbslot, 1]],
                                         rows4.at[1], gsem.at[1])
                else:
                    @pl.when(b + 1 < NBLK)
                    def _():
                        pltpu.async_copy(uc.at[sidx_b.at[nbslot, 2]],
                                         rows4.at[2], gsem.at[2])
            return 0
        lax.fori_loop(0, NBLK, bloop, 0)
        lastslot = (NBLK - 1) % 2
        pltpu.make_async_copy(
            rows4.at[3], acc_sh.at[didx_b.at[lastslot, 7]],
            ssem.at[3]).wait()

    def hop(h, _):
        _edge_sweep()
        plsc.subcore_barrier()
        _rescale(w_sh, False)
        plsc.subcore_barrier()
        return 0
    lax.fori_loop(0, K_HOPS - 1, hop, 0)
    _edge_sweep()
    plsc.subcore_barrier()
    _rescale(dd_sh, True)


_sc_call = functools.partial(
    pl.kernel,
    out_type=(jax.ShapeDtypeStruct((N, D), jnp.float32),
              jax.ShapeDtypeStruct((NC, NR, F), jnp.float32)),
    mesh=plsc.VectorSubcoreMesh(core_axis_name="c", subcore_axis_name="s",
                                num_cores=NC, num_subcores=NS),
    compiler_params=pltpu.CompilerParams(use_tc_tiling_on_sc=False),
    scratch_types=[
        pltpu.VMEM_SHARED((NR, F), jnp.float32),
        pltpu.VMEM_SHARED((NR,), jnp.float32),
        pltpu.VMEM_SHARED((NR,), jnp.float32),
        pltpu.VMEM_SHARED((NR,), jnp.float32),
        pltpu.VMEM((2, B, STEP), jnp.int32),
        pltpu.VMEM((2, B, STEP), jnp.int32),
        pltpu.VMEM((4, STEP, F), jnp.float32),
        pltpu.VMEM((STEP,), jnp.float32),
        pltpu.VMEM((2, CH, F), jnp.float32),
        pltpu.VMEM((CH, F), jnp.float32),
        pltpu.VMEM((2, CH), jnp.float32),
        pltpu.VMEM((RPT,), jnp.float32),
        pltpu.VMEM((RPT,), jnp.float32),
        pltpu.VMEM((RPT,), jnp.float32),
        pltpu.SemaphoreType.DMA((4,)),
        pltpu.SemaphoreType.DMA((4,)),
        pltpu.SemaphoreType.DMA,
        pltpu.SemaphoreType.DMA,
        pltpu.SemaphoreType.DMA((2,)),
        pltpu.SemaphoreType.DMA((2,)),
        pltpu.SemaphoreType.DMA((2,)),
        pltpu.SemaphoreType.DMA,
    ],
)(_sc_body)


@jax.jit
def kernel(x, edge_index, W1, b1, g1, be1, W2, b2, g2, be2):
    h0 = pl.pallas_call(
        _mlp_body,
        out_shape=jax.ShapeDtypeStruct((NC, NR, F), jnp.float32),
    )(x, W1, b1, g1, be1, W2, b2, g2, be2)

    pad = jnp.full((EP - E,), N, dtype=jnp.int32)
    srcp = jnp.concatenate([edge_index[0], pad]).reshape(NS, NBLK, B, STEP)
    dstp = jnp.concatenate([edge_index[1], pad]).reshape(NS, NBLK, B, STEP)

    out, _ = _sc_call(h0, srcp, dstp)
    return out


# speedup vs baseline: 2.7015x; 2.7015x over previous
import functools

import jax
import jax.numpy as jnp
from jax import lax
from jax.experimental import pallas as pl
from jax.experimental.pallas import tpu as pltpu
from jax.experimental.pallas import tpu_sc as plsc

N = 10000
D = 128
F = 64
K_HOPS = 10
NC = 2
NS = 16
NR = 10240
RPT = NR // NS
E = 320000
STEP = 128
B = 8
NBLK = 20
NSTEPS = NBLK * B
EP = NS * NSTEPS * STEP
CH = 32


def _mlp_body(x_ref, w1_ref, b1_ref, g1_ref, be1_ref,
              w2_ref, b2_ref, g2_ref, be2_ref, out_ref):
    eps = jnp.float32(1e-5)
    h = jnp.dot(x_ref[...], w1_ref[...],
                preferred_element_type=jnp.float32) + b1_ref[...]
    mean = jnp.mean(h, axis=0, keepdims=True)
    var = jnp.mean(jnp.square(h - mean), axis=0, keepdims=True)
    h = (h - mean) * lax.rsqrt(var + eps) * g1_ref[...] + be1_ref[...]
    h = jnp.maximum(h, 0.0)
    h = jnp.dot(h, w2_ref[...],
                preferred_element_type=jnp.float32) + b2_ref[...]
    mean = jnp.mean(h, axis=0, keepdims=True)
    var = jnp.mean(jnp.square(h - mean), axis=0, keepdims=True)
    h = (h - mean) * lax.rsqrt(var + eps) * g2_ref[...] + be2_ref[...]
    h = jnp.maximum(h, 0.0)
    zpad = jnp.zeros((NR - N, F), jnp.float32)
    out_ref[0, pl.ds(0, N), :] = h[:, :F]
    out_ref[1, pl.ds(0, N), :] = h[:, F:]
    out_ref[0, pl.ds(N, NR - N), :] = zpad
    out_ref[1, pl.ds(N, NR - N), :] = zpad


def _rsqrt16(x):
    i = lax.bitcast_convert_type(x, jnp.int32)
    i = jnp.int32(0x5F3759DF) - (i >> 1)
    y = lax.bitcast_convert_type(i, jnp.float32)
    for _ in range(3):
        y = y * (jnp.float32(1.5) - jnp.float32(0.5) * x * y * y)
    return y


def _sc_body(h0_hbm, srcp_hbm, dstp_hbm, out_hbm,
             u_sh, acc_sh, ds_sh, w_sh, dd_sh,
             sidx_b, didx_b, rows4, ones_v, rbuf2, zbuf, mbuf2,
             tb_a, tb_b, tb_c, gsem, ssem, isem, jsem,
             rsem, msem, usem, zsem):
    c = lax.axis_index("c")
    s = lax.axis_index("s")
    r0 = s * RPT
    uc = u_sh

    z16 = jnp.zeros((16,), jnp.float32)

    def zrow(i, _):
        for k4 in range(F // 16):
            zbuf[i, pl.ds(k4 * 16, 16)] = z16
        return 0
    lax.fori_loop(0, CH, zrow, 0)

    def zvec(i, _):
        tb_a[pl.ds(i * 16, 16)] = z16
        return 0
    lax.fori_loop(0, RPT // 16, zvec, 0)

    def o16(i, _):
        ones_v[pl.ds(i * 16, 16)] = jnp.ones((16,), jnp.float32)
        return 0
    lax.fori_loop(0, STEP // 16, o16, 0)

    pltpu.sync_copy(tb_a, ds_sh.at[pl.ds(r0, RPT)])
    pltpu.sync_copy(tb_a, w_sh.at[pl.ds(r0, RPT)])
    plsc.subcore_barrier()

    pltpu.sync_copy(srcp_hbm.at[s, 0], sidx_b.at[0])
    pltpu.sync_copy(dstp_hbm.at[s, 0], didx_b.at[0])

    def dblock(b, _):
        bslot = lax.rem(b, 2)
        nbslot = 1 - bslot

        @pl.when(b >= 1)
        def _():
            pltpu.make_async_copy(srcp_hbm.at[s, b], sidx_b.at[bslot],
                                  isem).wait()
            pltpu.make_async_copy(dstp_hbm.at[s, b], didx_b.at[bslot],
                                  jsem).wait()
            for j in range(B):
                pltpu.make_async_copy(
                    ones_v, ds_sh.at[sidx_b.at[nbslot, j]],
                    gsem.at[nbslot]).wait()
                pltpu.make_async_copy(
                    ones_v, w_sh.at[didx_b.at[nbslot, j]],
                    ssem.at[nbslot]).wait()

        @pl.when(b + 1 < NBLK)
        def _():
            pltpu.async_copy(srcp_hbm.at[s, b + 1], sidx_b.at[nbslot],
                             isem)
            pltpu.async_copy(dstp_hbm.at[s, b + 1], didx_b.at[nbslot],
                             jsem)

        for j in range(B):
            pltpu.async_copy(ones_v, ds_sh.at[sidx_b.at[bslot, j]],
                             gsem.at[bslot], add=True)
            pltpu.async_copy(ones_v, w_sh.at[didx_b.at[bslot, j]],
                             ssem.at[bslot], add=True)
        return 0
    lax.fori_loop(0, NBLK, dblock, 0)
    lastd = (NBLK - 1) % 2
    for j in range(B):
        pltpu.make_async_copy(
            ones_v, ds_sh.at[sidx_b.at[lastd, j]], gsem.at[lastd]).wait()
        pltpu.make_async_copy(
            ones_v, w_sh.at[didx_b.at[lastd, j]], ssem.at[lastd]).wait()

    pltpu.sync_copy(h0_hbm.at[c, pl.ds(r0, RPT), :],
                    acc_sh.at[pl.ds(r0, RPT), :])
    plsc.subcore_barrier()

    pltpu.sync_copy(ds_sh.at[pl.ds(r0, RPT)], tb_a)
    pltpu.sync_copy(w_sh.at[pl.ds(r0, RPT)], tb_b)
    def nloop(i, _):
        sl = pl.ds(i * 16, 16)
        xo = jnp.maximum(tb_a[sl], 1.0)
        xi = jnp.maximum(tb_b[sl], 1.0)
        ys = _rsqrt16(xo)
        yd = _rsqrt16(xi)
        tb_a[sl] = ys
        tb_b[sl] = ys * yd
        tb_c[sl] = yd
        return 0
    lax.fori_loop(0, RPT // 16, nloop, 0)
    pltpu.sync_copy(tb_a, ds_sh.at[pl.ds(r0, RPT)])
    pltpu.sync_copy(tb_b, w_sh.at[pl.ds(r0, RPT)])
    pltpu.sync_copy(tb_c, dd_sh.at[pl.ds(r0, RPT)])

    NCHK = RPT // CH

    def _rescale(mul_ref, to_hbm):

        def _compute(slot, cb):
            pltpu.sync_copy(acc_sh.at[pl.ds(cb, CH), :], rbuf2.at[slot])
            pltpu.sync_copy(mul_ref.at[pl.ds(cb, CH)], mbuf2.at[slot])

            def rgroup(g, _):
                mv = mbuf2[slot, pl.ds(g * 16, 16)]
                for r16 in range(16):
                    row = g * 16 + r16
                    m = mv[r16]
                    for k4 in range(F // 16):
                        sl = pl.ds(k4 * 16, 16)
                        rbuf2[slot, row, sl] = rbuf2[slot, row, sl] * m
                return 0
            lax.fori_loop(0, CH // 16, rgroup, 0)

        if to_hbm:
            def chunk(i, _):
                cb = r0 + i * CH
                _compute(0, cb)

                @pl.when(cb + CH <= N)
                def _():
                    pltpu.sync_copy(
                        rbuf2.at[0],
                        out_hbm.at[pl.ds(cb, CH), pl.ds(c * F, F)])

                @pl.when(cb == N - 16)
                def _():
                    pltpu.sync_copy(
                        rbuf2.at[0, pl.ds(0, 16), :],
                        out_hbm.at[pl.ds(cb, 16), pl.ds(c * F, F)])
                return 0
            lax.fori_loop(0, NCHK, chunk, 0)
        else:
            def chunk(ii, _):
                for half in range(2):
                    i = 2 * ii + half
                    cb = r0 + i * CH

                    @pl.when(i >= 2)
                    def _():
                        pltpu.make_async_copy(
                            rbuf2.at[half],
                            uc.at[pl.ds(cb - 2 * CH, CH), :],
                            usem.at[half]).wait()

                    _compute(half, cb)
                    pltpu.async_copy(rbuf2.at[half],
                                     uc.at[pl.ds(cb, CH), :],
                                     usem.at[half])
                    pltpu.async_copy(zbuf, acc_sh.at[pl.ds(cb, CH), :],
                                     zsem)
                return 0
            lax.fori_loop(0, NCHK // 2, chunk, 0)
            for half in range(2):
                pltpu.make_async_copy(
                    rbuf2.at[half],
                    uc.at[pl.ds(r0 + (NCHK - 2 + half) * CH, CH), :],
                    usem.at[half]).wait()

            def zdrain(i, _):
                pltpu.make_async_copy(
                    zbuf, acc_sh.at[pl.ds(r0, CH), :], zsem).wait()
                return 0
            lax.fori_loop(0, NCHK, zdrain, 0)

    _rescale(ds_sh, False)
    plsc.subcore_barrier()

    def _edge_sweep():
        pltpu.sync_copy(srcp_hbm.at[s, 0], sidx_b.at[0])
        pltpu.sync_copy(dstp_hbm.at[s, 0], didx_b.at[0])
        for q in range(3):
            pltpu.async_copy(uc.at[sidx_b.at[0, q]], rows4.at[q],
                             gsem.at[q])

        def bloop(b, _):
            bslot = lax.rem(b, 2)
            nbslot = 1 - bslot
            for j in range(B):
                q = j % 4
                pltpu.make_async_copy(
                    uc.at[sidx_b.at[bslot, j]], rows4.at[q],
                    gsem.at[q]).wait()
                pltpu.async_copy(rows4.at[q],
                                 acc_sh.at[didx_b.at[bslot, j]],
                                 ssem.at[q], add=True)
                if j == 0:
                    @pl.when(b >= 1)
                    def _():
                        pltpu.make_async_copy(
                            rows4.at[3], acc_sh.at[didx_b.at[nbslot, 7]],
                            ssem.at[3]).wait()

                    @pl.when(b + 1 < NBLK)
                    def _():
                        pltpu.async_copy(srcp_hbm.at[s, b + 1],
                                         sidx_b.at[nbslot], isem)
                        pltpu.async_copy(dstp_hbm.at[s, b + 1],
                                         didx_b.at[nbslot], jsem)
                else:
                    pltpu.make_async_copy(
                        rows4.at[(j - 1) % 4],
                        acc_sh.at[didx_b.at[bslot, j - 1]],
                        ssem.at[(j - 1) % 4]).wait()
                if j < 5:
                    pltpu.async_copy(uc.at[sidx_b.at[bslot, j + 3]],
                                     rows4.at[(j + 3) % 4],
                                     gsem.at[(j + 3) % 4])
                elif j == 5:
                    @pl.when(b + 1 < NBLK)
                    def _():
                        pltpu.make_async_copy(srcp_hbm.at[s, b + 1],
                                              sidx_b.at[nbslot],
                                              isem).wait()
                        pltpu.make_async_copy(dstp_hbm.at[s, b + 1],
                                              didx_b.at[nbslot],
                                              jsem).wait()
                        pltpu.async_copy(uc.at[sidx_b.at[nbslot, 0]],
                                         rows4.at[0], gsem.at[0])
                elif j == 6:
                    @pl.when(b + 1 < NBLK)
                    def _():
                        pltpu.async_copy(uc.at[sidx_b.at[nbslot, 1]],
                                         rows4.at[1], gsem.at[1])
                else:
                    @pl.when(b + 1 < NBLK)
                    def _():
                        pltpu.async_copy(uc.at[sidx_b.at[nbslot, 2]],
                                         rows4.at[2], gsem.at[2])
            return 0
        lax.fori_loop(0, NBLK, bloop, 0)
        lastslot = (NBLK - 1) % 2
        pltpu.make_async_copy(
            rows4.at[3], acc_sh.at[didx_b.at[lastslot, 7]],
            ssem.at[3]).wait()

    def hop(h, _):
        _edge_sweep()
        plsc.subcore_barrier()
        _rescale(w_sh, False)
        plsc.subcore_barrier()
        return 0
    lax.fori_loop(0, K_HOPS - 1, hop, 0)
    _edge_sweep()
    plsc.subcore_barrier()
    _rescale(dd_sh, True)


_sc_call = functools.partial(
    pl.kernel,
    out_type=jax.ShapeDtypeStruct((N, D), jnp.float32),
    mesh=plsc.VectorSubcoreMesh(core_axis_name="c", subcore_axis_name="s",
                                num_cores=NC, num_subcores=NS),
    compiler_params=pltpu.CompilerParams(use_tc_tiling_on_sc=False),
    scratch_types=[
        pltpu.VMEM_SHARED((NR, F), jnp.float32),
        pltpu.VMEM_SHARED((NR, F), jnp.float32),
        pltpu.VMEM_SHARED((NR,), jnp.float32),
        pltpu.VMEM_SHARED((NR,), jnp.float32),
        pltpu.VMEM_SHARED((NR,), jnp.float32),
        pltpu.VMEM((2, B, STEP), jnp.int32),
        pltpu.VMEM((2, B, STEP), jnp.int32),
        pltpu.VMEM((4, STEP, F), jnp.float32),
        pltpu.VMEM((STEP,), jnp.float32),
        pltpu.VMEM((2, CH, F), jnp.float32),
        pltpu.VMEM((CH, F), jnp.float32),
        pltpu.VMEM((2, CH), jnp.float32),
        pltpu.VMEM((RPT,), jnp.float32),
        pltpu.VMEM((RPT,), jnp.float32),
        pltpu.VMEM((RPT,), jnp.float32),
        pltpu.SemaphoreType.DMA((4,)),
        pltpu.SemaphoreType.DMA((4,)),
        pltpu.SemaphoreType.DMA,
        pltpu.SemaphoreType.DMA,
        pltpu.SemaphoreType.DMA((2,)),
        pltpu.SemaphoreType.DMA((2,)),
        pltpu.SemaphoreType.DMA((2,)),
        pltpu.SemaphoreType.DMA,
    ],
)(_sc_body)


@jax.jit
def kernel(x, edge_index, W1, b1, g1, be1, W2, b2, g2, be2):
    h0 = pl.pallas_call(
        _mlp_body,
        out_shape=jax.ShapeDtypeStruct((NC, NR, F), jnp.float32),
    )(x, W1, b1, g1, be1, W2, b2, g2, be2)

    pad = jnp.full((EP - E,), N, dtype=jnp.int32)
    srcp = jnp.concatenate([edge_index[0], pad]).reshape(NS, NBLK, B, STEP)
    dstp = jnp.concatenate([edge_index[1], pad]).reshape(NS, NBLK, B, STEP)

    return _sc_call(h0, srcp, dstp)
